# baseline (speedup 1.0000x reference)
import functools

import jax
import jax.numpy as jnp
from jax import lax
from jax.experimental import pallas as pl
from jax.experimental.pallas import tpu as pltpu
from jax.experimental.pallas import tpu_sc as plsc

N = 10000
E = 320000
NP = 10240
PT = NP // 16
CH = 128
TCH = 2560
EP = TCH * CH
CPT0 = 32
CPT1 = 128
BMAX = 128
K = 4

_mesh = plsc.VectorSubcoreMesh(core_axis_name="c", subcore_axis_name="s")
_sc_params = pltpu.CompilerParams(use_tc_tiling_on_sc=False)


def _chunk_start(c, s):
    return jnp.where(c == 0, s * CPT0, 16 * CPT0 + s * CPT1)


@functools.partial(
    pl.kernel,
    mesh=_mesh,
    out_type=jax.ShapeDtypeStruct((2, NP), jnp.float32),
    compiler_params=_sc_params,
    scratch_types=[
        pltpu.VMEM((BMAX, CH), jnp.int32),
        pltpu.VMEM((CH,), jnp.float32),
        pltpu.VMEM_SHARED((NP,), jnp.float32),
        pltpu.SemaphoreType.DMA,
        pltpu.SemaphoreType.DMA,
    ],
)
def _deg(dst_hbm, ones_hbm, z_hbm, out_hbm, dbuf, ones_v, acc, s0, s1):
    c = lax.axis_index("c")
    s = lax.axis_index("s")
    pltpu.sync_copy(z_hbm, acc.at[pl.ds(s * PT, PT)])
    pltpu.sync_copy(ones_hbm, ones_v)
    pltpu.sync_copy(dst_hbm.at[pl.ds(_chunk_start(c, s), BMAX)], dbuf)
    plsc.subcore_barrier()
    sems = [s0, s1]
    SW = 2 * K

    def fire_set(st, sem):
        for k in range(SW):
            pltpu.async_copy(ones_v, acc.at[dbuf.at[st * SW + k]], sem,
                             add=True)

    def drain_set(sem):
        for _ in range(SW):
            pltpu.make_async_copy(ones_v, acc.at[dbuf.at[0]], sem).wait()

    npairs = jnp.where(c == 0, CPT0 // SW // 2 - 1, CPT1 // SW // 2 - 1)
    fire_set(0, sems[0])
    fire_set(1, sems[1])

    def body(p, carry):
        drain_set(sems[0])
        fire_set(2 * p + 2, sems[0])
        drain_set(sems[1])
        fire_set(2 * p + 3, sems[1])
        return carry

    lax.fori_loop(0, npairs, body, 0)
    drain_set(sems[0])
    drain_set(sems[1])
    plsc.subcore_barrier()
    pltpu.sync_copy(acc.at[pl.ds(s * PT, PT)], out_hbm.at[c, pl.ds(s * PT, PT)])


def _make_agg(D, SK):
    rows_t = [pltpu.VMEM((CH, D), jnp.float32) for _ in range(2 * SK)]

    @functools.partial(
        pl.kernel,
        mesh=_mesh,
        out_type=jax.ShapeDtypeStruct((2, NP, D), jnp.float32),
        compiler_params=_sc_params,
        scratch_types=[
            pltpu.VMEM((BMAX, CH), jnp.int32),
            pltpu.VMEM((BMAX, CH), jnp.int32),
            pltpu.VMEM_SHARED((NP, D), jnp.float32),
            pltpu.SemaphoreType.DMA,
            pltpu.SemaphoreType.DMA,
            pltpu.SemaphoreType.DMA,
            pltpu.SemaphoreType.DMA,
        ]
        + rows_t,
    )
    def agg(src_hbm, dst_hbm, g_hbm, z_hbm, out_hbm, sbuf, dbuf, acc,
            sg0, sg1, ss0, ss1, *rows):
        c = lax.axis_index("c")
        s = lax.axis_index("s")
        start = _chunk_start(c, s)
        pltpu.sync_copy(z_hbm, acc.at[pl.ds(s * PT, PT)])
        pltpu.sync_copy(src_hbm.at[pl.ds(start, BMAX)], sbuf)
        pltpu.sync_copy(dst_hbm.at[pl.ds(start, BMAX)], dbuf)
        plsc.subcore_barrier()
        sg = [sg0, sg1]
        ss = [ss0, ss1]
        banks = [rows[0:SK], rows[SK:2 * SK]]

        def fire_gathers(st, bank, sem):
            for k in range(SK):
                pltpu.async_copy(g_hbm.at[sbuf.at[st * SK + k]], bank[k], sem)

        def wait_gathers(bank, sem):
            for k in range(SK):
                pltpu.make_async_copy(g_hbm.at[sbuf.at[0]], bank[k], sem).wait()

        def fire_scatters(st, bank, sem):
            for k in range(SK):
                pltpu.async_copy(bank[k], acc.at[dbuf.at[st * SK + k]], sem,
                                 add=True)

        def drain_scatters(bank, sem):
            for k in range(SK):
                pltpu.make_async_copy(bank[k], acc.at[dbuf.at[0]], sem).wait()

        def half(t, parity, first=False, last=False):
            X, Y = parity, 1 - parity
            if not last:
                if not first:
                    drain_scatters(banks[Y], ss[Y])
                fire_gathers(t + 1, banks[Y], sg[Y])
            wait_gathers(banks[X], sg[X])
            fire_scatters(t, banks[X], ss[X])

        ns0, ns1 = CPT0 // SK, CPT1 // SK
        ns = jnp.where(c == 0, ns0, ns1)
        npairs = jnp.where(c == 0, (ns0 - 2) // 2, (ns1 - 2) // 2)
        fire_gathers(0, banks[0], sg[0])
        half(0, 0, first=True)

        def body(p, carry):
            t = 2 * p + 1
            half(t, 1)
            half(t + 1, 0)
            return carry

        lax.fori_loop(0, npairs, body, 0)
        half(ns - 1, 1, last=True)
        drain_scatters(banks[0], ss[0])
        drain_scatters(banks[1], ss[1])
        plsc.subcore_barrier()
        pltpu.sync_copy(
            acc.at[pl.ds(s * PT, PT)], out_hbm.at[c, pl.ds(s * PT, PT)]
        )

    return agg


_agg64 = _make_agg(64, 2)
_agg32 = _make_agg(32, 4)


def _tc0(x, W1):
    def body(x_ref, w_ref, u_ref):
        u_ref[...] = jnp.dot(x_ref[...], w_ref[...],
                             preferred_element_type=jnp.float32)

    return pl.pallas_call(
        body,
        out_shape=jax.ShapeDtypeStruct((N, 64), jnp.float32),
    )(x, W1)


def _tc1(u1, degT):
    def body(u_ref, dg_ref, g1_ref, dinv_ref):
        dsum = dg_ref[:, 0:1] + dg_ref[:, 1:2] + 1.0
        dinv = lax.rsqrt(dsum[0:N, :])
        g1_ref[...] = u_ref[...] * dinv
        dinv_ref[...] = dinv

    return pl.pallas_call(
        body,
        out_shape=(
            jax.ShapeDtypeStruct((N, 64), jnp.float32),
            jax.ShapeDtypeStruct((N, 1), jnp.float32),
        ),
    )(u1, degT)


def _tc2(p0, p1, g1, dinv, W2, b1):
    def body(p0_ref, p1_ref, g1_ref, dinv_ref, w_ref, b_ref, g2_ref):
        agg = p0_ref[0:N, :] + p1_ref[0:N, :] + g1_ref[...]
        h1 = jnp.maximum(dinv_ref[...] * agg + b_ref[...], 0.0)
        u = jnp.dot(h1, w_ref[...], preferred_element_type=jnp.float32)
        g2_ref[...] = u * dinv_ref[...]

    return pl.pallas_call(
        body,
        out_shape=jax.ShapeDtypeStruct((N, 32), jnp.float32),
    )(p0, p1, g1, dinv, W2, b1)


def _tc3(q0, q1, g2, dinv, w3r, b2, b3):
    def body(q0_ref, q1_ref, g2_ref, dinv_ref, w_ref, b2_ref, b3_ref, o_ref):
        agg = q0_ref[0:N, :] + q1_ref[0:N, :] + g2_ref[...]
        h2 = jnp.maximum(dinv_ref[...] * agg + b2_ref[...], 0.0)
        o_ref[...] = (
            jnp.sum(h2 * w_ref[...], axis=1, keepdims=True) + b3_ref[...]
        )

    return pl.pallas_call(
        body,
        out_shape=jax.ShapeDtypeStruct((N, 1), jnp.float32),
    )(q0, q1, g2, dinv, w3r, b2, b3)


def kernel(x, edge_index, W1, b1, W2, b2, W3, b3):
    src = edge_index[0].astype(jnp.int32)
    dst = edge_index[1].astype(jnp.int32)
    pad = EP - E
    src_p = jnp.concatenate([src, jnp.zeros((pad,), jnp.int32)])
    dst_p = jnp.concatenate([dst, jnp.full((pad,), N, jnp.int32)])
    src2 = src_p.reshape(TCH, CH)
    dst2 = dst_p.reshape(TCH, CH)
    ones = jnp.ones((CH,), jnp.float32)
    z1 = jnp.zeros((PT,), jnp.float32)
    z64 = jnp.zeros((PT, 64), jnp.float32)
    z32 = jnp.zeros((PT, 32), jnp.float32)

    degp = _deg(dst2, ones, z1)
    u1 = _tc0(x, W1)
    g1, dinv = _tc1(u1, degp.T)
    p = _agg64(src2, dst2, g1, z64)
    g2 = _tc2(p[0], p[1], g1, dinv, W2, b1.reshape(1, 64))
    q = _agg32(src2, dst2, g2, z32)
    out = _tc3(q[0], q[1], g2, dinv, W3.reshape(1, 32), b2.reshape(1, 32),
               b3.reshape(1, 1))
    return out


# speedup vs baseline: 2.0399x; 2.0399x over previous
import functools

import jax
import jax.numpy as jnp
from jax import lax
from jax.experimental import pallas as pl
from jax.experimental.pallas import tpu as pltpu
from jax.experimental.pallas import tpu_sc as plsc

N = 10000
E = 320000
NP = 10240
PT = NP // 16
CH = 128
TCH = 2560
EP = TCH * CH
CPT0 = 80
CPT1 = 80
BMAX = 80
K = 4
NA = 10112
PA = NA // 16

_mesh = plsc.VectorSubcoreMesh(core_axis_name="c", subcore_axis_name="s")
_sc_params = pltpu.CompilerParams(use_tc_tiling_on_sc=False)


def _chunk_start(c, s):
    return jnp.where(c == 0, s * CPT0, 16 * CPT0 + s * CPT1)


@functools.partial(
    pl.kernel,
    mesh=_mesh,
    out_type=jax.ShapeDtypeStruct((2, NP), jnp.float32),
    compiler_params=_sc_params,
    scratch_types=[
        pltpu.VMEM((BMAX, CH), jnp.int32),
        pltpu.VMEM((CH,), jnp.float32),
        pltpu.VMEM_SHARED((NP,), jnp.float32),
        pltpu.SemaphoreType.DMA,
        pltpu.SemaphoreType.DMA,
    ],
)
def _deg(dst_hbm, ones_hbm, z_hbm, out_hbm, dbuf, ones_v, acc, s0, s1):
    c = lax.axis_index("c")
    s = lax.axis_index("s")
    pltpu.sync_copy(z_hbm, acc.at[pl.ds(s * PT, PT)])
    pltpu.sync_copy(ones_hbm, ones_v)
    pltpu.sync_copy(dst_hbm.at[pl.ds(_chunk_start(c, s), BMAX)], dbuf)
    plsc.subcore_barrier()
    sems = [s0, s1]
    SW = 2 * K

    def fire_set(st, sem):
        for k in range(SW):
            pltpu.async_copy(ones_v, acc.at[dbuf.at[st * SW + k]], sem,
                             add=True)

    def drain_set(sem):
        for _ in range(SW):
            pltpu.make_async_copy(ones_v, acc.at[dbuf.at[0]], sem).wait()

    npairs = jnp.where(c == 0, CPT0 // SW // 2 - 1, CPT1 // SW // 2 - 1)
    fire_set(0, sems[0])
    fire_set(1, sems[1])

    def body(p, carry):
        drain_set(sems[0])
        fire_set(2 * p + 2, sems[0])
        drain_set(sems[1])
        fire_set(2 * p + 3, sems[1])
        return carry

    lax.fori_loop(0, npairs, body, 0)
    drain_set(sems[0])
    drain_set(sems[1])
    plsc.subcore_barrier()
    pltpu.sync_copy(acc.at[pl.ds(s * PT, PT)], out_hbm.at[c, pl.ds(s * PT, PT)])


def _make_agg(D, SK):
    rows_t = [pltpu.VMEM((CH, D), jnp.float32) for _ in range(2 * SK)]

    @functools.partial(
        pl.kernel,
        mesh=_mesh,
        out_type=jax.ShapeDtypeStruct((2, NA, D), jnp.float32),
        compiler_params=_sc_params,
        scratch_types=[
            pltpu.VMEM((BMAX, CH), jnp.int32),
            pltpu.VMEM((BMAX, CH), jnp.int32),
            pltpu.VMEM_SHARED((NA, D), jnp.float32),
            pltpu.VMEM_SHARED((NA, D), jnp.float32),
            pltpu.SemaphoreType.DMA,
            pltpu.SemaphoreType.DMA,
            pltpu.SemaphoreType.DMA,
            pltpu.SemaphoreType.DMA,
        ]
        + rows_t,
    )
    def agg(src_hbm, dst_hbm, g_hbm, z_hbm, out_hbm, sbuf, dbuf, acc, tbl,
            sg0, sg1, ss0, ss1, *rows):
        c = lax.axis_index("c")
        s = lax.axis_index("s")
        start = _chunk_start(c, s)
        pltpu.sync_copy(z_hbm, acc.at[pl.ds(s * PA, PA)])
        pltpu.sync_copy(g_hbm.at[pl.ds(s * PA, PA)], tbl.at[pl.ds(s * PA, PA)])
        pltpu.sync_copy(src_hbm.at[pl.ds(start, BMAX)], sbuf)
        pltpu.sync_copy(dst_hbm.at[pl.ds(start, BMAX)], dbuf)
        plsc.subcore_barrier()
        sg = [sg0, sg1]
        ss = [ss0, ss1]
        banks = [rows[0:SK], rows[SK:2 * SK]]

        def fire_gathers(st, bank, sem):
            for k in range(SK):
                pltpu.async_copy(tbl.at[sbuf.at[st * SK + k]], bank[k], sem)

        def wait_gathers(bank, sem):
            for k in range(SK):
                pltpu.make_async_copy(tbl.at[sbuf.at[0]], bank[k], sem).wait()

        def fire_scatters(st, bank, sem):
            for k in range(SK):
                pltpu.async_copy(bank[k], acc.at[dbuf.at[st * SK + k]], sem,
                                 add=True)

        def drain_scatters(bank, sem):
            for k in range(SK):
                pltpu.make_async_copy(bank[k], acc.at[dbuf.at[0]], sem).wait()

        def half(t, parity, first=False, last=False):
            X, Y = parity, 1 - parity
            if not last:
                if not first:
                    drain_scatters(banks[Y], ss[Y])
                fire_gathers(t + 1, banks[Y], sg[Y])
            wait_gathers(banks[X], sg[X])
            fire_scatters(t, banks[X], ss[X])

        ns0, ns1 = CPT0 // SK, CPT1 // SK
        ns = jnp.where(c == 0, ns0, ns1)
        npairs = jnp.where(c == 0, (ns0 - 2) // 2, (ns1 - 2) // 2)
        fire_gathers(0, banks[0], sg[0])
        half(0, 0, first=True)

        def body(p, carry):
            t = 2 * p + 1
            half(t, 1)
            half(t + 1, 0)
            return carry

        lax.fori_loop(0, npairs, body, 0)
        half(ns - 1, 1, last=True)
        drain_scatters(banks[0], ss[0])
        drain_scatters(banks[1], ss[1])
        plsc.subcore_barrier()
        pltpu.sync_copy(
            acc.at[pl.ds(s * PA, PA)], out_hbm.at[c, pl.ds(s * PA, PA)]
        )

    return agg


_agg64 = _make_agg(64, 1)
_agg32 = _make_agg(32, 4)


def _tc0(x, W1):
    def body(x_ref, w_ref, u_ref):
        u_ref[...] = jnp.dot(x_ref[...], w_ref[...],
                             preferred_element_type=jnp.float32)

    return pl.pallas_call(
        body,
        out_shape=jax.ShapeDtypeStruct((N, 64), jnp.float32),
    )(x, W1)


def _tc1(u1, degT):
    def body(u_ref, dg_ref, g1_ref, dinv_ref):
        dsum = dg_ref[:, 0:1] + dg_ref[:, 1:2] + 1.0
        dinv = lax.rsqrt(dsum[0:N, :])
        g1_ref[...] = u_ref[...] * dinv
        dinv_ref[...] = dinv

    return pl.pallas_call(
        body,
        out_shape=(
            jax.ShapeDtypeStruct((N, 64), jnp.float32),
            jax.ShapeDtypeStruct((N, 1), jnp.float32),
        ),
    )(u1, degT)


def _tc2(p0, p1, g1, dinv, W2, b1):
    def body(p0_ref, p1_ref, g1_ref, dinv_ref, w_ref, b_ref, g2_ref):
        agg = p0_ref[0:N, :] + p1_ref[0:N, :] + g1_ref[...]
        h1 = jnp.maximum(dinv_ref[...] * agg + b_ref[...], 0.0)
        u = jnp.dot(h1, w_ref[...], preferred_element_type=jnp.float32)
        g2_ref[...] = u * dinv_ref[...]

    return pl.pallas_call(
        body,
        out_shape=jax.ShapeDtypeStruct((N, 32), jnp.float32),
    )(p0, p1, g1, dinv, W2, b1)


def _tc3(q0, q1, g2, dinv, w3r, b2, b3):
    def body(q0_ref, q1_ref, g2_ref, dinv_ref, w_ref, b2_ref, b3_ref, o_ref):
        agg = q0_ref[0:N, :] + q1_ref[0:N, :] + g2_ref[...]
        h2 = jnp.maximum(dinv_ref[...] * agg + b2_ref[...], 0.0)
        o_ref[...] = (
            jnp.sum(h2 * w_ref[...], axis=1, keepdims=True) + b3_ref[...]
        )

    return pl.pallas_call(
        body,
        out_shape=jax.ShapeDtypeStruct((N, 1), jnp.float32),
    )(q0, q1, g2, dinv, w3r, b2, b3)


def kernel(x, edge_index, W1, b1, W2, b2, W3, b3):
    src = edge_index[0].astype(jnp.int32)
    dst = edge_index[1].astype(jnp.int32)
    pad = EP - E
    src_p = jnp.concatenate([src, jnp.zeros((pad,), jnp.int32)])
    dst_p = jnp.concatenate([dst, jnp.full((pad,), N, jnp.int32)])
    src2 = src_p.reshape(TCH, CH)
    dst2 = dst_p.reshape(TCH, CH)
    ones = jnp.ones((CH,), jnp.float32)
    z1 = jnp.zeros((PT,), jnp.float32)
    z64 = jnp.zeros((PA, 64), jnp.float32)
    z32 = jnp.zeros((PA, 32), jnp.float32)

    degp = _deg(dst2, ones, z1)
    u1 = _tc0(x, W1)
    g1, dinv = _tc1(u1, degp.T)
    g1p = jnp.concatenate([g1, jnp.zeros((NA - N, 64), jnp.float32)])
    p = _agg64(src2, dst2, g1p, z64)
    g2 = _tc2(p[0], p[1], g1, dinv, W2, b1.reshape(1, 64))
    g2p = jnp.concatenate([g2, jnp.zeros((NA - N, 32), jnp.float32)])
    q = _agg32(src2, dst2, g2p, z32)
    out = _tc3(q[0], q[1], g2, dinv, W3.reshape(1, 32), b2.reshape(1, 32),
               b3.reshape(1, 1))
    return out
